# baseline (speedup 1.0000x reference)
import jax
import jax.numpy as jnp
from jax.experimental import pallas as pl
from jax.experimental.pallas import tpu as pltpu

_N = 10000
_D = 512
_BM = 400
_BMH = 200
_NMF = (_N - _BM) // _BM
_NM = _NMF + 2
_SC = 2000
_NSUP = _N // _SC


def _row0(j):
    return jnp.where(j < _NMF, j * _BM, _NMF * _BM + (j - _NMF) * _BMH)


def _full_copy(adj_hbm, buf_ref, sem, j, slot):
    return pltpu.make_async_copy(
        adj_hbm.at[pl.ds(_row0(j), _BM), :],
        buf_ref.at[slot],
        sem.at[slot],
    )


def _half_copy(adj_hbm, buf_ref, sem, j, slot):
    return pltpu.make_async_copy(
        adj_hbm.at[pl.ds(_row0(j), _BMH), :],
        buf_ref.at[slot, pl.ds(0, _BMH), :],
        sem.at[slot],
    )


def _start(adj_hbm, buf_ref, sem, j, slot):
    @pl.when(j < _NMF)
    def _f():
        _full_copy(adj_hbm, buf_ref, sem, j, slot).start()

    @pl.when(jnp.logical_and(j >= _NMF, j < _NM))
    def _h():
        _half_copy(adj_hbm, buf_ref, sem, j, slot).start()


def _stage2(o, gamma_ref, beta_ref):
    mean = jnp.mean(o, axis=-1, keepdims=True)
    var = jnp.mean(jnp.square(o - mean), axis=-1, keepdims=True)
    o = (o - mean) * jax.lax.rsqrt(var + 1e-5) * gamma_ref[...] + beta_ref[...]
    return jnp.maximum(o, 0.0)


def _fused_body(inp_ref, w_ref, w2_ref, gamma_ref, beta_ref, adj_hbm,
                out_ref, sup_ref, buf_ref, sem):
    i = pl.program_id(0)

    @pl.when(i == 0)
    def _kickoff():
        _full_copy(adj_hbm, buf_ref, sem, 0, 0).start()
        _full_copy(adj_hbm, buf_ref, sem, 1, 1).start()

    @pl.when(i < _NSUP)
    def _prologue():
        chunk = jnp.dot(inp_ref[...], w_ref[...],
                        preferred_element_type=jnp.float32)
        sup_ref[pl.ds(i * _SC, _SC), :] = chunk.astype(jnp.bfloat16)

    @pl.when(i >= _NSUP)
    def _main():
        j = i - _NSUP
        slot = jax.lax.rem(j, 2)

        @pl.when(j < _NMF)
        def _full_step():
            _full_copy(adj_hbm, buf_ref, sem, j, slot).wait()
            a = buf_ref[slot].astype(jnp.bfloat16)
            h = jnp.dot(a, sup_ref[...], preferred_element_type=jnp.float32)
            h = jnp.maximum(h, 0.0).astype(jnp.bfloat16)
            o = jnp.dot(h, w2_ref[...], preferred_element_type=jnp.float32)
            out_ref[...] = _stage2(o, gamma_ref, beta_ref)

        @pl.when(j >= _NMF)
        def _half_step():
            _half_copy(adj_hbm, buf_ref, sem, j, slot).wait()
            a = buf_ref[slot, pl.ds(0, _BMH), :].astype(jnp.bfloat16)
            h = jnp.dot(a, sup_ref[...], preferred_element_type=jnp.float32)
            h = jnp.maximum(h, 0.0).astype(jnp.bfloat16)
            o = jnp.dot(h, w2_ref[...], preferred_element_type=jnp.float32)
            out_ref[pl.ds((j - _NMF) * _BMH, _BMH), :] = _stage2(
                o, gamma_ref, beta_ref)

        _start(adj_hbm, buf_ref, sem, j + 2, slot)


def kernel(input, adj, weight, weight2, gamma, beta):
    w_bf16 = weight.astype(jnp.bfloat16)
    w2_bf16 = weight2.astype(jnp.bfloat16)
    gamma2d = gamma.reshape(1, _D)
    beta2d = beta.reshape(1, _D)

    def _out_idx(i):
        return (jnp.minimum(jnp.maximum(i - _NSUP, 0), _NMF), 0)

    out = pl.pallas_call(
        _fused_body,
        grid=(_NSUP + _NM,),
        in_specs=[
            pl.BlockSpec((_SC, _D), lambda i: (jnp.minimum(i, _NSUP - 1), 0)),
            pl.BlockSpec((_D, _D), lambda i: (0, 0)),
            pl.BlockSpec((_D, _D), lambda i: (0, 0)),
            pl.BlockSpec((1, _D), lambda i: (0, 0)),
            pl.BlockSpec((1, _D), lambda i: (0, 0)),
            pl.BlockSpec(memory_space=pl.ANY),
        ],
        out_specs=pl.BlockSpec((_BM, _D), _out_idx),
        out_shape=jax.ShapeDtypeStruct((_N, _D), jnp.float32),
        scratch_shapes=[
            pltpu.VMEM((_N, _D), jnp.bfloat16),
            pltpu.VMEM((2, _BM, _N), jnp.float32),
            pltpu.SemaphoreType.DMA((2,)),
        ],
        compiler_params=pltpu.CompilerParams(
            dimension_semantics=("arbitrary",),
        ),
    )(input, w_bf16, w2_bf16, gamma2d, beta2d, adj)
    return out
